# baseline (speedup 1.0000x reference)
import jax
import jax.numpy as jnp
from jax import lax
from jax.experimental import pallas as pl
from jax.experimental.pallas import tpu as pltpu
from jax.experimental.pallas import tpu_sc as plsc

B, C, H, W = 8, 3, 512, 512
PATCH = 16
GY, GX = H // PATCH, W // PATCH
NSEG_PER_IMG = GY * GX
NV = B * NSEG_PER_IMG
N = B * H * W
HW = H * W

NC, NS, L = 2, 16, 16
NW = NC * NS
PER_W = N // NW
VCH = 4096
RCH = VCH // W
NV_W = NV // NW


def _bbox_tc_kernel(bbox_ref):
    jb = jax.lax.broadcasted_iota(jnp.int32, (4, NV), 0)
    v = jax.lax.broadcasted_iota(jnp.int32, (4, NV), 1)
    off = jax.lax.broadcasted_iota(jnp.int32, (PATCH, PATCH), 0)
    omin = jnp.min(off)
    omax = jnp.max(off)
    py = (v % NSEG_PER_IMG) // GX
    px = v % GX
    bbox_ref[...] = jnp.where(
        jb == 0, py * PATCH + omin,
        jnp.where(jb == 1, px * PATCH + omin,
                  jnp.where(jb == 2, py * PATCH + omax,
                            px * PATCH + omax)))


def _sc_body(img_hbm, fv_hbm, seg_hbm, byx_hbm, xin_v, fo_v, sg_v, b3_v):
    wid = lax.axis_index("s") * NC + lax.axis_index("c")
    base = wid * PER_W
    iota = lax.iota(jnp.int32, L)

    def chunk(k, carry):
        n0 = pl.multiple_of(base + k * VCH, VCH)
        b = n0 >> 18
        h0 = (n0 >> 9) & (H - 1)
        for c in range(C):
            src_row = pl.multiple_of(b * (C * H) + c * H + h0, RCH)
            pltpu.sync_copy(
                img_hbm.at[pl.ds(src_row, RCH), :],
                xin_v.at[pl.ds(c * RCH, RCH), :])

        def vec(t, carry2):
            row = t >> 5
            col = (t & 31) * L
            prow = pl.ds(t * L, L)
            prel = t * L + iota
            for c in range(C):
                v_c = xin_v[c * RCH + row, pl.ds(col, L)]
                plsc.store_scatter(fo_v, [prel * 3 + c], v_c)
            n = n0 + prel
            bb = n >> 18
            yy = (n >> 9) & (W - 1)
            xx = n & (W - 1)
            sg_v[prow] = (bb << 10) | ((yy >> 4) << 5) | (xx >> 4)
            b3_v[0, prow] = bb
            b3_v[1, prow] = yy
            b3_v[2, prow] = xx
            return carry2

        lax.fori_loop(0, VCH // L, vec, 0)
        pltpu.sync_copy(fo_v, fv_hbm.at[pl.ds(n0 * 3, VCH * 3)])
        pltpu.sync_copy(sg_v, seg_hbm.at[pl.ds(n0, VCH)])
        pltpu.sync_copy(b3_v, byx_hbm.at[:, pl.ds(n0, VCH)])
        return carry

    lax.fori_loop(0, PER_W // VCH, chunk, 0)


_sc_call = pl.kernel(
    _sc_body,
    out_type=[
        jax.ShapeDtypeStruct((N * C,), jnp.float32),
        jax.ShapeDtypeStruct((N,), jnp.int32),
        jax.ShapeDtypeStruct((3, N), jnp.int32),
    ],
    mesh=plsc.VectorSubcoreMesh(core_axis_name="c", subcore_axis_name="s"),
    compiler_params=pltpu.CompilerParams(needs_layout_passes=False),
    scratch_types=[
        pltpu.VMEM((C * RCH, W), jnp.float32),
        pltpu.VMEM((VCH * C,), jnp.float32),
        pltpu.VMEM((VCH,), jnp.int32),
        pltpu.VMEM((3, VCH), jnp.int32),
    ],
)


def kernel(img):
    img2 = img.reshape(B * C * H, W)
    fv_flat, seg, byx = _sc_call(img2)
    fV = fv_flat.reshape(N, C)
    bbox = pl.pallas_call(
        _bbox_tc_kernel,
        grid=(1,),
        in_specs=[],
        out_specs=pl.BlockSpec((4, NV), lambda i: (0, 0)),
        out_shape=jax.ShapeDtypeStruct((4, NV), jnp.int32),
    )()
    return (fV, seg, byx, bbox)


# speedup vs baseline: 1.6142x; 1.6142x over previous
import jax
import jax.numpy as jnp
from jax import lax
from jax.experimental import pallas as pl
from jax.experimental.pallas import tpu as pltpu
from jax.experimental.pallas import tpu_sc as plsc

B, C, H, W = 8, 3, 512, 512
PATCH = 16
GY, GX = H // PATCH, W // PATCH
NSEG_PER_IMG = GY * GX
NV = B * NSEG_PER_IMG
N = B * H * W
HB = 64
PIXB = HB * W

NC, NS, L = 2, 16, 16
NW = NC * NS
PER_W = N // NW
VCH = 8192
NV_W = NV // NW


def _fv_kernel(img_ref, fv_ref):
    x = img_ref[0]
    fv_ref[...] = jnp.transpose(x, (1, 2, 0)).reshape(PIXB, C)


def _idx_sc_body(seg_hbm, byx_hbm, bbox_hbm, sg_v, b3_v, bx_v):
    wid = lax.axis_index("s") * NC + lax.axis_index("c")
    base = wid * PER_W

    def chunk(k, carry):
        n0 = base + k * VCH

        def vec(t, carry2):
            n = n0 + t * L + lax.iota(jnp.int32, L)
            bb = n >> 18
            yy = (n >> 9) & (W - 1)
            xx = n & (W - 1)
            sl = pl.ds(t * L, L)
            sg_v[sl] = (bb << 10) | ((yy >> 4) << 5) | (xx >> 4)
            b3_v[0, sl] = bb
            b3_v[1, sl] = yy
            b3_v[2, sl] = xx
            return carry2

        lax.fori_loop(0, VCH // L, vec, 0)
        pltpu.sync_copy(sg_v, seg_hbm.at[pl.ds(n0, VCH)])
        pltpu.sync_copy(b3_v, byx_hbm.at[:, pl.ds(n0, VCH)])
        return carry

    lax.fori_loop(0, PER_W // VCH, chunk, 0)

    v0 = wid * NV_W

    def bvec(t, carry):
        v = v0 + t * L + lax.iota(jnp.int32, L)
        py = (v >> 5) & (GY - 1)
        px = v & (GX - 1)
        sl = pl.ds(t * L, L)
        bx_v[0, sl] = py * PATCH
        bx_v[1, sl] = px * PATCH
        bx_v[2, sl] = py * PATCH + (PATCH - 1)
        bx_v[3, sl] = px * PATCH + (PATCH - 1)
        return carry

    lax.fori_loop(0, NV_W // L, bvec, 0)
    pltpu.sync_copy(bx_v, bbox_hbm.at[:, pl.ds(v0, NV_W)])


_idx_sc = pl.kernel(
    _idx_sc_body,
    out_type=[
        jax.ShapeDtypeStruct((N,), jnp.int32),
        jax.ShapeDtypeStruct((3, N), jnp.int32),
        jax.ShapeDtypeStruct((4, NV), jnp.int32),
    ],
    mesh=plsc.VectorSubcoreMesh(core_axis_name="c", subcore_axis_name="s"),
    scratch_types=[
        pltpu.VMEM((VCH,), jnp.int32),
        pltpu.VMEM((3, VCH), jnp.int32),
        pltpu.VMEM((4, NV_W), jnp.int32),
    ],
)


def kernel(img):
    blocks_per_img = H // HB
    fV = pl.pallas_call(
        _fv_kernel,
        grid=(N // PIXB,),
        in_specs=[
            pl.BlockSpec((1, C, HB, W),
                         lambda i: (i // blocks_per_img, 0, i % blocks_per_img, 0)),
        ],
        out_specs=pl.BlockSpec((PIXB, C), lambda i: (i, 0)),
        out_shape=jax.ShapeDtypeStruct((N, C), jnp.float32),
    )(img)

    seg, byx, bbox = _idx_sc()
    return (fV, seg, byx, bbox)
